# Initial kernel scaffold; baseline (speedup 1.0000x reference)
import jax
import jax.numpy as jnp
from jax.experimental import pallas as pl


def kernel(x, edge_index, edge_attr, W_node, b_node, W_edge, b_edge, W_msg, b_msg, W_upd, b_upd, W1, b1, W2, b2, W3, b3):
    raise NotImplementedError("write your pallas kernel here")


# speedup vs baseline: 3.0497x; 3.0497x over previous
import functools

import jax
import jax.numpy as jnp
from jax import lax
from jax.experimental import pallas as pl
from jax.experimental.pallas import tpu as pltpu
from jax.experimental.pallas import tpu_sc as plsc

N = 10000
E = 320000
D = 128
L = 3
NC = 2
NS = 16
NW = NC * NS
EPW = E // NW
K = 80
NCH = EPW // K
RPT = 624
NSL = D // 16


def _sc_edge_layer(nfp, src, dst, ea2, Ci, dvi):
    mesh = plsc.VectorSubcoreMesh(core_axis_name="c", subcore_axis_name="s")

    @functools.partial(
        pl.kernel,
        out_type=jax.ShapeDtypeStruct((NC, N, D), jnp.float32),
        mesh=mesh,
        scratch_types=[
            pltpu.VMEM((K,), jnp.int32),
            pltpu.VMEM((K,), jnp.int32),
            pltpu.VMEM((2 * K,), jnp.float32),
            pltpu.VMEM((K, D), jnp.float32),
            pltpu.VMEM((2, D), jnp.float32),
            pltpu.VMEM((D,), jnp.float32),
            pltpu.VMEM_SHARED((N, D), jnp.float32),
            pltpu.SemaphoreType.DMA,
        ],
    )
    def k(nfp_h, src_h, dst_h, ea2_h, c_h, dv_h, out_h,
          src_v, dst_v, attr_v, rows_v, c_v, dv_v, agg_s, sem):
        cid = lax.axis_index("c")
        sid = lax.axis_index("s")
        wid = cid * NS + sid
        base = wid * EPW

        pltpu.sync_copy(c_h, c_v)
        pltpu.sync_copy(dv_h, dv_v)

        def zrow(i, _):
            for s in range(NSL):
                rows_v[i, pl.ds(s * 16, 16)] = jnp.zeros((16,), jnp.float32)
            return 0
        lax.fori_loop(0, K, zrow, 0)
        row0 = sid * RPT

        def zero_stripe(r0, nrows):
            nfull = nrows // K
            for o in range(nfull):
                pltpu.sync_copy(rows_v, agg_s.at[pl.ds(r0 + o * K, K)])
            rem = nrows - nfull * K
            if rem:
                pltpu.sync_copy(rows_v.at[pl.ds(0, rem)],
                                agg_s.at[pl.ds(r0 + nfull * K, rem)])
        zero_stripe(row0, RPT)

        @pl.when(sid == NS - 1)
        def _():
            zero_stripe(NS * RPT, N - NS * RPT)
        plsc.subcore_barrier()

        c0 = [c_v[0, pl.ds(s * 16, 16)] for s in range(NSL)]
        c1 = [c_v[1, pl.ds(s * 16, 16)] for s in range(NSL)]
        dv = [dv_v[pl.ds(s * 16, 16)] for s in range(NSL)]

        def chunk_body(c, _):
            off = base + c * K
            pltpu.sync_copy(src_h.at[pl.ds(off, K)], src_v)
            pltpu.sync_copy(dst_h.at[pl.ds(off, K)], dst_v)
            pltpu.sync_copy(ea2_h.at[pl.ds(2 * off, 2 * K)], attr_v)
            pltpu.async_copy(nfp_h.at[src_v], rows_v, sem).wait()

            def group_body(g, _):
                av = attr_v[pl.ds(g * 16, 16)]
                for e in range(8):
                    j = g * 8 + e
                    a0 = av[2 * e]
                    a1 = av[2 * e + 1]
                    for s in range(NSL):
                        t = a1 * c1[s] + dv[s]
                        t = a0 * c0[s] + t
                        r = rows_v[j, pl.ds(s * 16, 16)]
                        rows_v[j, pl.ds(s * 16, 16)] = jnp.maximum(r + t, 0.0)
                return 0
            lax.fori_loop(0, K // 8, group_body, 0)
            pltpu.sync_copy(rows_v, agg_s.at[dst_v], add=True)
            return 0
        lax.fori_loop(0, NCH, chunk_body, 0)
        plsc.subcore_barrier()

        pltpu.sync_copy(agg_s.at[pl.ds(row0, RPT)],
                        out_h.at[cid, pl.ds(row0, RPT)])

        @pl.when(sid == NS - 1)
        def _():
            pltpu.sync_copy(agg_s.at[pl.ds(NS * RPT, N - NS * RPT)],
                            out_h.at[cid, pl.ds(NS * RPT, N - NS * RPT)])

    return k(nfp, src, dst, ea2, Ci, dvi)


def _tc_pre(x, W_node, b_node, W_edge, b_edge, W_msg, b_msg):
    def body(x_r, wn_r, bn_r, we_r, be_r, wm_r, bm_r,
             nf_o, nfp_o, c_o, dv_o):
        nf = jnp.dot(x_r[...], wn_r[...],
                     preferred_element_type=jnp.float32) + bn_r[...]
        nf_o[...] = nf
        nfp_o[...] = jnp.dot(nf, wm_r[0, :D, :],
                             preferred_element_type=jnp.float32)
        for i in range(L):
            wb = wm_r[i, D:, :]
            c_o[i] = jnp.dot(we_r[...], wb,
                             preferred_element_type=jnp.float32)
            dv_o[pl.ds(i, 1), :] = (jnp.dot(be_r[...], wb,
                                            preferred_element_type=jnp.float32)
                                    + bm_r[pl.ds(i, 1), :])
    return pl.pallas_call(
        body,
        out_shape=(
            jax.ShapeDtypeStruct((N, D), jnp.float32),
            jax.ShapeDtypeStruct((N, D), jnp.float32),
            jax.ShapeDtypeStruct((L, 2, D), jnp.float32),
            jax.ShapeDtypeStruct((L, D), jnp.float32),
        ),
    )(x, W_node, b_node.reshape(1, D), W_edge, b_edge.reshape(1, D),
      W_msg, b_msg)


def _tc_update(parts, nf, Wu, bu, Wt):
    def body(p_r, nf_r, wu_r, bu_r, wt_r, nf_o, nfp_o):
        agg = p_r[0] + p_r[1]
        u = jnp.maximum(
            jnp.dot(agg, wu_r[...], preferred_element_type=jnp.float32)
            + bu_r[...], 0.0)
        nfn = u + nf_r[...]
        nf_o[...] = nfn
        nfp_o[...] = jnp.dot(nfn, wt_r[...],
                             preferred_element_type=jnp.float32)
    return pl.pallas_call(
        body,
        out_shape=(
            jax.ShapeDtypeStruct((N, D), jnp.float32),
            jax.ShapeDtypeStruct((N, D), jnp.float32),
        ),
    )(parts, nf, Wu, bu.reshape(1, D), Wt)


def _tc_final(parts, nf, Wu, bu, W1, b1, W2, b2, W3, b3):
    def body(p_r, nf_r, wu_r, bu_r, w1_r, b1_r, w2_r, b2_r, w3_r, b3_r,
             out_o):
        agg = p_r[0] + p_r[1]
        u = jnp.maximum(
            jnp.dot(agg, wu_r[...], preferred_element_type=jnp.float32)
            + bu_r[...], 0.0)
        nfn = u + nf_r[...]
        h = jnp.mean(nfn, axis=0, keepdims=True)
        h = jnp.maximum(
            jnp.dot(h, w1_r[...], preferred_element_type=jnp.float32)
            + b1_r[...], 0.0)
        h = jnp.maximum(
            jnp.dot(h, w2_r[...], preferred_element_type=jnp.float32)
            + b2_r[...], 0.0)
        out_o[...] = (jnp.dot(h, w3_r[...],
                              preferred_element_type=jnp.float32)
                      + b3_r[...])
    return pl.pallas_call(
        body,
        out_shape=jax.ShapeDtypeStruct((1, 1), jnp.float32),
    )(parts, nf, Wu, bu.reshape(1, D), W1, b1.reshape(1, D // 2),
      W2, b2.reshape(1, D // 2), W3, b3.reshape(1, 1))


def kernel(x, edge_index, edge_attr, W_node, b_node, W_edge, b_edge,
           W_msg, b_msg, W_upd, b_upd, W1, b1, W2, b2, W3, b3):
    src = edge_index[0]
    dst = edge_index[1]
    ea2 = edge_attr[:, :2].reshape(-1)

    nf, nfp, C, dv = _tc_pre(x, W_node, b_node, W_edge, b_edge, W_msg, b_msg)
    for i in range(L):
        parts = _sc_edge_layer(nfp, src, dst, ea2, C[i], dv[i])
        if i < L - 1:
            nf, nfp = _tc_update(parts, nf, W_upd[i], b_upd[i],
                                 W_msg[i + 1, :D, :])
        else:
            pred = _tc_final(parts, nf, W_upd[i], b_upd[i],
                             W1, b1, W2, b2, W3, b3)
    return pred


# speedup vs baseline: 4.3458x; 1.4250x over previous
import functools

import jax
import jax.numpy as jnp
from jax import lax
from jax.experimental import pallas as pl
from jax.experimental.pallas import tpu as pltpu
from jax.experimental.pallas import tpu_sc as plsc

N = 10000
E = 320000
D = 128
L = 3
NC = 2
NS = 16
NW = NC * NS
EPW = E // NW
K = 40
NCH = EPW // K
NB = 5
GA = 3
RPT = 624
NSL = D // 16


def _sc_edge_layer(nfp, src, dst, ea2, Ci):
    mesh = plsc.VectorSubcoreMesh(core_axis_name="c", subcore_axis_name="s")

    @functools.partial(
        pl.kernel,
        out_type=jax.ShapeDtypeStruct((NC, N, D), jnp.float32),
        mesh=mesh,
        scratch_types=[
            pltpu.VMEM((NB, K), jnp.int32),
            pltpu.VMEM((NB, K), jnp.int32),
            pltpu.VMEM((NB, 2 * K), jnp.float32),
            pltpu.VMEM((2, D), jnp.float32),
            pltpu.VMEM_SHARED((N, D), jnp.float32),
        ] + [pltpu.VMEM((K, D), jnp.float32) for _ in range(NB)]
          + [pltpu.SemaphoreType.DMA for _ in range(3 * NB)],
    )
    def k(nfp_h, src_h, dst_h, ea2_h, c_h, out_h,
          src_v, dst_v, attr_v, c_v, agg_s, *rest):
        rows = list(rest[:NB])
        gsem = list(rest[NB:2 * NB])
        ssem = list(rest[2 * NB:3 * NB])
        isem = list(rest[3 * NB:])
        cid = lax.axis_index("c")
        sid = lax.axis_index("s")
        wid = cid * NS + sid

        pltpu.sync_copy(c_h, c_v)

        def idx_load(ch, b, sem):
            pltpu.async_copy(src_h.at[wid, ch], src_v.at[b], sem)
            pltpu.async_copy(dst_h.at[wid, ch], dst_v.at[b], sem)
            pltpu.async_copy(ea2_h.at[wid, ch], attr_v.at[b], sem)

        def idx_wait(b, sem):
            pltpu.make_async_copy(src_h.at[0, 0], src_v.at[b], sem).wait()
            pltpu.make_async_copy(dst_h.at[0, 0], dst_v.at[b], sem).wait()
            pltpu.make_async_copy(ea2_h.at[0, 0], attr_v.at[b], sem).wait()

        def zrow(i, _):
            for s in range(NSL):
                rows[0][i, pl.ds(s * 16, 16)] = jnp.zeros((16,), jnp.float32)
            return 0
        lax.fori_loop(0, K, zrow, 0)
        row0 = sid * RPT

        def zero_stripe(r0, nrows):
            nfull = nrows // K
            for o in range(nfull):
                pltpu.sync_copy(rows[0], agg_s.at[pl.ds(r0 + o * K, K)])
            rem = nrows - nfull * K
            if rem:
                pltpu.sync_copy(rows[0].at[pl.ds(0, rem)],
                                agg_s.at[pl.ds(r0 + nfull * K, rem)])
        zero_stripe(row0, RPT)

        @pl.when(sid == NS - 1)
        def _():
            zero_stripe(NS * RPT, N - NS * RPT)
        plsc.subcore_barrier()

        c0 = [c_v[0, pl.ds(s * 16, 16)] for s in range(NSL)]
        c1 = [c_v[1, pl.ds(s * 16, 16)] for s in range(NSL)]

        for ch in range(GA + 1):
            idx_load(ch, ch, isem[ch])
        for g in range(GA):
            idx_wait(g, isem[g])
            pltpu.async_copy(nfp_h.at[src_v.at[g]], rows[g], gsem[g])

        def compute_chunk(b, rbuf):
            def group_body(gi, _):
                av = attr_v[b, pl.ds(gi * 16, 16)]
                for e in range(8):
                    j = gi * 8 + e
                    a0 = av[2 * e]
                    a1 = av[2 * e + 1]
                    for s in range(NSL):
                        t = a0 * c0[s] + a1 * c1[s]
                        r = rbuf[j, pl.ds(s * 16, 16)]
                        rbuf[j, pl.ds(s * 16, 16)] = jnp.maximum(r + t, 0.0)
                return 0
            lax.fori_loop(0, K // 8, group_body, 0)

        def outer_body(o, _):
            for b in range(NB):
                c = o * NB + b
                g = c + GA
                gb = (b + GA) % NB
                h = c + GA + 1
                hb = (b + GA + 1) % NB

                @pl.when(g < NCH)
                def _issue():
                    idx_wait(gb, isem[gb])
                    pltpu.async_copy(nfp_h.at[src_v.at[gb]], rows[gb],
                                     gsem[gb])

                pltpu.make_async_copy(
                    nfp_h.at[src_v.at[0]], rows[b], gsem[b]).wait()
                compute_chunk(b, rows[b])
                pltpu.async_copy(rows[b], agg_s.at[dst_v.at[b]], ssem[b],
                                 add=True)

                @pl.when(h < NCH)
                def _prefetch():
                    @pl.when(h >= NB)
                    def _():
                        pltpu.make_async_copy(
                            rows[hb], agg_s.at[dst_v.at[0]], ssem[hb]).wait()
                    idx_load(h, hb, isem[hb])
            return 0
        lax.fori_loop(0, NCH // NB, outer_body, 0)

        for b in range(NB):
            pltpu.make_async_copy(
                rows[b], agg_s.at[dst_v.at[0]], ssem[b]).wait()
        plsc.subcore_barrier()

        pltpu.sync_copy(agg_s.at[pl.ds(row0, RPT)],
                        out_h.at[cid, pl.ds(row0, RPT)])

        @pl.when(sid == NS - 1)
        def _():
            pltpu.sync_copy(agg_s.at[pl.ds(NS * RPT, N - NS * RPT)],
                            out_h.at[cid, pl.ds(NS * RPT, N - NS * RPT)])

    return k(nfp, src, dst, ea2, Ci)


def _tc_pre(x, W_node, b_node, W_edge, b_edge, W_msg, b_msg):
    def body(x_r, wn_r, bn_r, we_r, be_r, wm_r, bm_r,
             nf_o, nfp_o, c_o, dv_o):
        for i in range(L):
            wb = wm_r[i, D:, :]
            c_o[i] = jnp.dot(we_r[...], wb,
                             preferred_element_type=jnp.float32,
                  precision=lax.Precision.HIGHEST)
            dv_o[pl.ds(i, 1), :] = (jnp.dot(be_r[...], wb,
                                            preferred_element_type=jnp.float32,
                  precision=lax.Precision.HIGHEST)
                                    + bm_r[pl.ds(i, 1), :])
        nf = jnp.dot(x_r[...], wn_r[...],
                     preferred_element_type=jnp.float32,
                  precision=lax.Precision.HIGHEST) + bn_r[...]
        nf_o[...] = nf
        nfp_o[...] = jnp.dot(nf, wm_r[0, :D, :],
                             preferred_element_type=jnp.float32,
                  precision=lax.Precision.HIGHEST) + dv_o[0, :]
    return pl.pallas_call(
        body,
        out_shape=(
            jax.ShapeDtypeStruct((N, D), jnp.float32),
            jax.ShapeDtypeStruct((N, D), jnp.float32),
            jax.ShapeDtypeStruct((L, 2, D), jnp.float32),
            jax.ShapeDtypeStruct((L, D), jnp.float32),
        ),
    )(x, W_node, b_node.reshape(1, D), W_edge, b_edge.reshape(1, D),
      W_msg, b_msg)


def _tc_update(parts, nf, Wu, bu, Wt, dvn):
    def body(p_r, nf_r, wu_r, bu_r, wt_r, dv_r, nf_o, nfp_o):
        agg = p_r[0] + p_r[1]
        u = jnp.maximum(
            jnp.dot(agg, wu_r[...], preferred_element_type=jnp.float32,
                  precision=lax.Precision.HIGHEST)
            + bu_r[...], 0.0)
        nfn = u + nf_r[...]
        nf_o[...] = nfn
        nfp_o[...] = jnp.dot(nfn, wt_r[...],
                             preferred_element_type=jnp.float32,
                  precision=lax.Precision.HIGHEST) + dv_r[...]
    return pl.pallas_call(
        body,
        out_shape=(
            jax.ShapeDtypeStruct((N, D), jnp.float32),
            jax.ShapeDtypeStruct((N, D), jnp.float32),
        ),
    )(parts, nf, Wu, bu.reshape(1, D), Wt, dvn)


def _tc_final(parts, nf, Wu, bu, W1, b1, W2, b2, W3, b3):
    def body(p_r, nf_r, wu_r, bu_r, w1_r, b1_r, w2_r, b2_r, w3_r, b3_r,
             out_o):
        agg = p_r[0] + p_r[1]
        u = jnp.maximum(
            jnp.dot(agg, wu_r[...], preferred_element_type=jnp.float32,
                  precision=lax.Precision.HIGHEST)
            + bu_r[...], 0.0)
        nfn = u + nf_r[...]
        h = jnp.mean(nfn, axis=0, keepdims=True)
        h = jnp.maximum(
            jnp.dot(h, w1_r[...], preferred_element_type=jnp.float32,
                  precision=lax.Precision.HIGHEST)
            + b1_r[...], 0.0)
        h = jnp.maximum(
            jnp.dot(h, w2_r[...], preferred_element_type=jnp.float32,
                  precision=lax.Precision.HIGHEST)
            + b2_r[...], 0.0)
        out_o[...] = (jnp.dot(h, w3_r[...],
                              preferred_element_type=jnp.float32,
                  precision=lax.Precision.HIGHEST)
                      + b3_r[...])
    return pl.pallas_call(
        body,
        out_shape=jax.ShapeDtypeStruct((1, 1), jnp.float32),
    )(parts, nf, Wu, bu.reshape(1, D), W1, b1.reshape(1, D // 2),
      W2, b2.reshape(1, D // 2), W3, b3.reshape(1, 1))


def kernel(x, edge_index, edge_attr, W_node, b_node, W_edge, b_edge,
           W_msg, b_msg, W_upd, b_upd, W1, b1, W2, b2, W3, b3):
    src = edge_index[0].reshape(NW, NCH, K)
    dst = edge_index[1].reshape(NW, NCH, K)
    ea2 = edge_attr[:, :2].reshape(NW, NCH, 2 * K)

    nf, nfp, C, dv = _tc_pre(x, W_node, b_node, W_edge, b_edge, W_msg, b_msg)
    for i in range(L):
        parts = _sc_edge_layer(nfp, src, dst, ea2, C[i])
        if i < L - 1:
            nf, nfp = _tc_update(parts, nf, W_upd[i], b_upd[i],
                                 W_msg[i + 1, :D, :], dv[i + 1:i + 2])
        else:
            pred = _tc_final(parts, nf, W_upd[i], b_upd[i],
                             W1, b1, W2, b2, W3, b3)
    return pred


# speedup vs baseline: 5.2749x; 1.2138x over previous
import functools

import jax
import jax.numpy as jnp
from jax import lax
from jax.experimental import pallas as pl
from jax.experimental.pallas import tpu as pltpu
from jax.experimental.pallas import tpu_sc as plsc

N = 10000
E = 320000
D = 128
L = 3
NC = 2
NS = 16
NW = NC * NS
EPW = E // NW
K = 40
NCH = EPW // K
NB = 5
NBI = 10
GA = 3
RPT = 624

def _sc_edge_layer(nfp, src, dst, ea2, Ci):
    mesh = plsc.VectorSubcoreMesh(core_axis_name="c", subcore_axis_name="s")

    @functools.partial(
        pl.kernel,
        out_type=jax.ShapeDtypeStruct((NC, N, D), jnp.float32),
        mesh=mesh,
        scratch_types=[
            pltpu.VMEM((NBI, K), jnp.int32),
            pltpu.VMEM((NBI, K), jnp.int32),
            pltpu.VMEM((NBI, 2 * K), jnp.float32),
            pltpu.VMEM((2, D), jnp.float32),
            pltpu.VMEM_SHARED((N, D), jnp.float32),
        ] + [pltpu.VMEM((K, D), jnp.float32) for _ in range(NB)]
          + [pltpu.SemaphoreType.DMA for _ in range(2 * NB + NBI)],
    )
    def k(nfp_h, src_h, dst_h, ea2_h, c_h, out_h,
          src_v, dst_v, attr_v, c_v, agg_s, *rest):
        rows = list(rest[:NB])
        gsem = list(rest[NB:2 * NB])
        ssem = list(rest[2 * NB:3 * NB])
        isem = list(rest[3 * NB:])
        cid = lax.axis_index("c")
        sid = lax.axis_index("s")
        wid = cid * NS + sid

        pltpu.sync_copy(c_h, c_v)

        def idx_load(ch, b):
            pltpu.async_copy(src_h.at[wid, ch], src_v.at[b], isem[b])
            pltpu.async_copy(dst_h.at[wid, ch], dst_v.at[b], isem[b])
            pltpu.async_copy(ea2_h.at[wid, ch], attr_v.at[b], isem[b])

        def idx_wait(b):
            pltpu.make_async_copy(src_h.at[0, 0], src_v.at[b], isem[b]).wait()
            pltpu.make_async_copy(dst_h.at[0, 0], dst_v.at[b], isem[b]).wait()
            pltpu.make_async_copy(ea2_h.at[0, 0], attr_v.at[b], isem[b]).wait()

        def zrow(i, _):
            for s in range(D // 16):
                rows[0][i, pl.ds(s * 16, 16)] = jnp.zeros((16,), jnp.float32)
            return 0
        lax.fori_loop(0, K, zrow, 0)
        row0 = sid * RPT

        def zero_stripe(r0, nrows):
            nfull = nrows // K
            for o in range(nfull):
                pltpu.sync_copy(rows[0], agg_s.at[pl.ds(r0 + o * K, K)])
            rem = nrows - nfull * K
            if rem:
                pltpu.sync_copy(rows[0].at[pl.ds(0, rem)],
                                agg_s.at[pl.ds(r0 + nfull * K, rem)])
        zero_stripe(row0, RPT)

        @pl.when(sid == NS - 1)
        def _():
            zero_stripe(NS * RPT, N - NS * RPT)
        plsc.subcore_barrier()

        c0 = [c_v[0, pl.ds(s * 16, 16)] for s in range(D // 16)]
        c1 = [c_v[1, pl.ds(s * 16, 16)] for s in range(D // 16)]

        for ch in range(GA + 1):
            idx_load(ch, ch)
        for g in range(GA):
            idx_wait(g)
            pltpu.async_copy(nfp_h.at[src_v.at[g]], rows[g], gsem[g])

        def compute_chunk(t, rb):
            def group_body(gi, _):
                av = attr_v[t, pl.ds(gi * 16, 16)]
                for e in range(8):
                    j = gi * 8 + e
                    a0 = av[2 * e]
                    a1 = av[2 * e + 1]
                    for s in range(D // 16):
                        tt = a0 * c0[s] + a1 * c1[s]
                        r = rows[rb][j, pl.ds(s * 16, 16)]
                        rows[rb][j, pl.ds(s * 16, 16)] = (
                            jnp.maximum(r + tt, 0.0))
                return 0
            lax.fori_loop(0, K // 8, group_body, 0)

        def outer_body(o, _):
            for t in range(NBI):
                c = o * NBI + t
                rb = t % NB
                g = c + GA
                gb = (t + GA) % NB
                gi = (t + GA) % NBI
                h = c + GA + 1
                hi = (t + GA + 1) % NBI

                @pl.when(g < NCH)
                def _issue():
                    idx_wait(gi)
                    @pl.when(g >= NB)
                    def _():
                        pltpu.make_async_copy(
                            rows[gb], agg_s.at[dst_v.at[0]], ssem[gb]).wait()
                    pltpu.async_copy(nfp_h.at[src_v.at[gi]], rows[gb],
                                     gsem[gb])

                @pl.when(h < NCH)
                def _prefetch():
                    idx_load(h, hi)

                pltpu.make_async_copy(
                    nfp_h.at[src_v.at[0]], rows[rb], gsem[rb]).wait()

                compute_chunk(t, rb)
                pltpu.async_copy(rows[rb], agg_s.at[dst_v.at[t]], ssem[rb],
                                 add=True)
            return 0
        lax.fori_loop(0, NCH // NBI, outer_body, 0)

        for b in range(NB):
            pltpu.make_async_copy(
                rows[b], agg_s.at[dst_v.at[0]], ssem[b]).wait()
        plsc.subcore_barrier()

        pltpu.sync_copy(agg_s.at[pl.ds(row0, RPT)],
                        out_h.at[cid, pl.ds(row0, RPT)])

        @pl.when(sid == NS - 1)
        def _():
            pltpu.sync_copy(agg_s.at[pl.ds(NS * RPT, N - NS * RPT)],
                            out_h.at[cid, pl.ds(NS * RPT, N - NS * RPT)])

    return k(nfp, src, dst, ea2, Ci)


def _tc_pre(x, W_node, b_node, W_edge, b_edge, W_msg, b_msg):
    def body(x_r, wn_r, bn_r, we_r, be_r, wm_r, bm_r,
             nf_o, nfp_o, c_o, dv_o):
        for i in range(L):
            wb = wm_r[i, D:, :]
            c_o[i] = jnp.dot(we_r[...], wb,
                             preferred_element_type=jnp.float32,
                             precision=lax.Precision.HIGHEST)
            dv_o[pl.ds(i, 1), :] = (jnp.dot(be_r[...], wb,
                                            preferred_element_type=jnp.float32,
                                            precision=lax.Precision.HIGHEST)
                                    + bm_r[pl.ds(i, 1), :])
        nf = jnp.dot(x_r[...], wn_r[...],
                     preferred_element_type=jnp.float32,
                     precision=lax.Precision.HIGHEST) + bn_r[...]
        nf_o[...] = nf
        nfp_o[...] = (jnp.dot(nf, wm_r[0, :D, :],
                              preferred_element_type=jnp.float32,
                              precision=lax.Precision.HIGHEST)
                      + dv_o[0, :])
    return pl.pallas_call(
        body,
        out_shape=(
            jax.ShapeDtypeStruct((N, D), jnp.float32),
            jax.ShapeDtypeStruct((N, D), jnp.float32),
            jax.ShapeDtypeStruct((L, 2, D), jnp.float32),
            jax.ShapeDtypeStruct((L, D), jnp.float32),
        ),
    )(x, W_node, b_node.reshape(1, D), W_edge, b_edge.reshape(1, D),
      W_msg, b_msg)


def _tc_update(parts, nf, Wup, bu, Wt, dvn):
    def body(p_r, nf_r, wu_r, bu_r, wt_r, dv_r, nf_o, nfp_o):
        agg = p_r[0] + p_r[1]
        u = jnp.maximum(
            jnp.dot(agg, wu_r[...], preferred_element_type=jnp.float32,
                    precision=lax.Precision.HIGHEST)
            + bu_r[...], 0.0)
        nfn = u + nf_r[...]
        nf_o[...] = nfn
        nfp_o[...] = (jnp.dot(nfn, wt_r[...],
                              preferred_element_type=jnp.float32,
                              precision=lax.Precision.HIGHEST)
                      + dv_r[...])
    return pl.pallas_call(
        body,
        out_shape=(
            jax.ShapeDtypeStruct((N, D), jnp.float32),
            jax.ShapeDtypeStruct((N, D), jnp.float32),
        ),
    )(parts, nf, Wup, bu.reshape(1, D), Wt, dvn)


def _tc_final(parts, nf, Wup, bu, W1, b1, W2, b2, W3, b3):
    def body(p_r, nf_r, wu_r, bu_r, w1_r, b1_r, w2_r, b2_r, w3_r, b3_r,
             out_o):
        agg = p_r[0] + p_r[1]
        u = jnp.maximum(
            jnp.dot(agg, wu_r[...], preferred_element_type=jnp.float32,
                    precision=lax.Precision.HIGHEST)
            + bu_r[...], 0.0)
        nfn = u + nf_r[...]
        h = jnp.mean(nfn, axis=0, keepdims=True)
        h = jnp.maximum(
            jnp.dot(h, w1_r[...], preferred_element_type=jnp.float32,
                    precision=lax.Precision.HIGHEST)
            + b1_r[...], 0.0)
        h = jnp.maximum(
            jnp.dot(h, w2_r[...], preferred_element_type=jnp.float32,
                    precision=lax.Precision.HIGHEST)
            + b2_r[...], 0.0)
        out_o[...] = (jnp.dot(h, w3_r[...],
                              preferred_element_type=jnp.float32,
                              precision=lax.Precision.HIGHEST)
                      + b3_r[...])
    return pl.pallas_call(
        body,
        out_shape=jax.ShapeDtypeStruct((1, 1), jnp.float32),
    )(parts, nf, Wup, bu.reshape(1, D), W1, b1.reshape(1, D // 2),
      W2, b2.reshape(1, D // 2), W3, b3.reshape(1, 1))


def kernel(x, edge_index, edge_attr, W_node, b_node, W_edge, b_edge,
           W_msg, b_msg, W_upd, b_upd, W1, b1, W2, b2, W3, b3):
    src = edge_index[0].reshape(NW, NCH, K)
    dst = edge_index[1].reshape(NW, NCH, K)
    ea2 = edge_attr[:, :2].reshape(NW, NCH, 2 * K)
    nf, nfp, C, dv = _tc_pre(x, W_node, b_node, W_edge, b_edge, W_msg, b_msg)
    for i in range(L):
        parts = _sc_edge_layer(nfp, src, dst, ea2, C[i])
        Wup = W_upd[i]
        if i < L - 1:
            nf, nfp = _tc_update(parts, nf, Wup, b_upd[i],
                                 W_msg[i + 1, :D, :], dv[i + 1:i + 2])
        else:
            pred = _tc_final(parts, nf, Wup, b_upd[i],
                             W1, b1, W2, b2, W3, b3)
    return pred
